# baseline (speedup 1.0000x reference)
import functools

import jax
import jax.numpy as jnp
from jax.experimental import pallas as pl
from jax.experimental.pallas import tpu as pltpu

_CB = 128


def _top2(probs, tile, ne):
    eidx = jax.lax.broadcasted_iota(jnp.int32, (tile, ne), 1)
    p0 = jnp.max(probs, axis=-1, keepdims=True)
    e0 = jnp.min(jnp.where(probs == p0, eidx, ne), axis=-1, keepdims=True)
    masked = jnp.where(eidx == e0, -jnp.inf, probs)
    p1 = jnp.max(masked, axis=-1, keepdims=True)
    e1 = jnp.min(jnp.where(masked == p1, eidx, ne), axis=-1, keepdims=True)
    return p0, e0, p1, e1


def _router_kernel(x_ref, w1_ref, b1_ref, w2_ref, b2_ref,
                   disp_ref, comb_ref, probs_ref, aux_ref,
                   probs_s, c0_s, sums_s,
                   *, tile, tiles, ne, cap, ntok):
    p = pl.program_id(0)
    t = pl.program_id(1)
    c = pl.program_id(2)
    cblocks = cap // _CB
    eidx = jax.lax.broadcasted_iota(jnp.int32, (tile, ne), 1)

    @pl.when(p == 0)
    def _pass0():
        disp_ref[...] = jnp.zeros((tile, ne, _CB), jnp.float32)
        comb_ref[...] = jnp.zeros((tile, ne, _CB), jnp.float32)

        @pl.when(c == 0)
        def _router():
            x = x_ref[...]
            h = jnp.maximum(
                jnp.dot(x, w1_ref[...], preferred_element_type=jnp.float32)
                + b1_ref[...], 0.0)
            logits = (jnp.dot(h, w2_ref[...],
                              preferred_element_type=jnp.float32)
                      + b2_ref[...])
            m = jnp.max(logits, axis=-1, keepdims=True)
            ex = jnp.exp(logits - m)
            probs = ex / jnp.sum(ex, axis=-1, keepdims=True)
            probs_s[pl.ds(t * tile, tile), :] = probs

            _, e0, _, _ = _top2(probs, tile, ne)
            flags = jnp.max((eidx == e0).astype(jnp.float32), axis=0,
                            keepdims=True)
            psum = jnp.sum(probs, axis=0, keepdims=True)
            first = t == 0
            c0_s[...] = jnp.where(first, flags,
                                  jnp.maximum(c0_s[...], flags))
            sums_s[...] = jnp.where(first, psum, sums_s[...] + psum)

            @pl.when(t == tiles - 1)
            def _aux():
                mean = sums_s[...] * (1.0 / ntok)
                aux_ref[...] = jnp.sum(mean * jnp.log(mean * ne + 1e-09),
                                       axis=-1, keepdims=True)

    @pl.when((p == 1) & (c == 0))
    def _pass1():
        probs = probs_s[pl.ds(t * tile, tile), :]
        p0, e0, p1, e1 = _top2(probs, tile, ne)
        s = p0 + p1
        p0n = p0 / s
        p1n = p1 / s
        c0 = c0_s[...]
        slot1 = jnp.sum(jnp.where(eidx == e1, c0, 0.0), axis=-1,
                        keepdims=True).astype(jnp.int32)
        slotmat = jnp.where(eidx == e0, 0,
                            jnp.where(eidx == e1, slot1, -1))
        valmat = jnp.where(eidx == e0, p0n,
                           jnp.where(eidx == e1, p1n, 0.0))
        siota = jax.lax.broadcasted_iota(jnp.int32, (tile, ne, _CB), 2)
        hit = siota == slotmat[:, :, None]
        disp_ref[...] = hit.astype(jnp.float32)
        comb_ref[...] = jnp.where(hit, valmat[:, :, None], 0.0)
        probs_ref[...] = probs


def kernel(hidden_states, W1, b1, W2, b2):
    bb, ss, hh = hidden_states.shape
    ne = W2.shape[1]
    ntok = bb * ss
    cap = int(bb * ss * 1.5 * 2 / ne)
    x = hidden_states.reshape(ntok, hh)
    b1r = b1.reshape(1, hh)
    b2r = b2.reshape(1, ne)
    tile = 256
    tiles = ntok // tile
    cblocks = cap // _CB

    body = functools.partial(_router_kernel, tile=tile, tiles=tiles,
                             ne=ne, cap=cap, ntok=ntok)

    def _bigmap(p, t, c):
        return (t, 0, jnp.where(p == 0, jnp.maximum(c, 1), 0))

    disp, comb, probs, aux = pl.pallas_call(
        body,
        grid=(2, tiles, cblocks),
        in_specs=[
            pl.BlockSpec((tile, hh),
                         lambda p, t, c: (jnp.where(p == 0, t, 0), 0)),
            pl.BlockSpec((hh, hh), lambda p, t, c: (0, 0)),
            pl.BlockSpec((1, hh), lambda p, t, c: (0, 0)),
            pl.BlockSpec((hh, ne), lambda p, t, c: (0, 0)),
            pl.BlockSpec((1, ne), lambda p, t, c: (0, 0)),
        ],
        out_specs=[
            pl.BlockSpec((tile, ne, _CB), _bigmap),
            pl.BlockSpec((tile, ne, _CB), _bigmap),
            pl.BlockSpec((tile, ne),
                         lambda p, t, c: (jnp.where(p == 1, t, 0), 0)),
            pl.BlockSpec((1, 1), lambda p, t, c: (0, 0)),
        ],
        out_shape=[
            jax.ShapeDtypeStruct((ntok, ne, cap), jnp.float32),
            jax.ShapeDtypeStruct((ntok, ne, cap), jnp.float32),
            jax.ShapeDtypeStruct((ntok, ne), jnp.float32),
            jax.ShapeDtypeStruct((1, 1), jnp.float32),
        ],
        scratch_shapes=[
            pltpu.VMEM((ntok, ne), jnp.float32),
            pltpu.VMEM((1, ne), jnp.float32),
            pltpu.VMEM((1, ne), jnp.float32),
        ],
        compiler_params=pltpu.CompilerParams(
            dimension_semantics=("arbitrary", "arbitrary", "arbitrary")),
    )(x, W1, b1r, W2, b2r)

    return (disp.reshape(bb, ss, ne, cap),
            comb.reshape(bb, ss, ne, cap),
            probs.reshape(bb, ss, ne),
            aux.reshape(()))


# speedup vs baseline: 1.7229x; 1.7229x over previous
import functools

import jax
import jax.numpy as jnp
from jax.experimental import pallas as pl
from jax.experimental.pallas import tpu as pltpu

_LEAD = 128


def _top2(probs, tile, ne):
    eidx = jax.lax.broadcasted_iota(jnp.int32, (tile, ne), 1)
    p0 = jnp.max(probs, axis=-1, keepdims=True)
    e0 = jnp.min(jnp.where(probs == p0, eidx, ne), axis=-1, keepdims=True)
    masked = jnp.where(eidx == e0, -jnp.inf, probs)
    p1 = jnp.max(masked, axis=-1, keepdims=True)
    e1 = jnp.min(jnp.where(masked == p1, eidx, ne), axis=-1, keepdims=True)
    return p0, e0, p1, e1


def _router_kernel(x_ref, w1_ref, b1_ref, w2_ref, b2_ref,
                   disp_ref, comb_ref, probs_ref, aux_ref,
                   probs_s, c0_s, sums_s, zbuf, dbuf, cbuf, sem_z, sem_c,
                   *, tile, tiles, ne, cap, ntok):
    p = pl.program_id(0)
    t = pl.program_id(1)
    eidx = jax.lax.broadcasted_iota(jnp.int32, (tile, ne), 1)
    ztail = cap - _LEAD

    def _zcopy(dst_ref, row, qi):
        return pltpu.make_async_copy(
            zbuf, dst_ref.at[pl.ds(row, tile), :, pl.ds(_LEAD, ztail)],
            sem_z.at[qi])

    @pl.when(p == 0)
    def _pass0():
        @pl.when(t == 0)
        def _init():
            zbuf[...] = jnp.zeros_like(zbuf)

        _zcopy(disp_ref, t * tile, (2 * t) % 4).start()
        _zcopy(comb_ref, t * tile, (2 * t + 1) % 4).start()

        x = x_ref[...]
        h = jnp.maximum(
            jnp.dot(x, w1_ref[...], preferred_element_type=jnp.float32)
            + b1_ref[...], 0.0)
        logits = (jnp.dot(h, w2_ref[...], preferred_element_type=jnp.float32)
                  + b2_ref[...])
        m = jnp.max(logits, axis=-1, keepdims=True)
        ex = jnp.exp(logits - m)
        probs = ex / jnp.sum(ex, axis=-1, keepdims=True)
        probs_s[pl.ds(t * tile, tile), :] = probs

        _, e0, _, _ = _top2(probs, tile, ne)
        flags = jnp.max((eidx == e0).astype(jnp.float32), axis=0,
                        keepdims=True)
        psum = jnp.sum(probs, axis=0, keepdims=True)
        first = t == 0
        c0_s[...] = jnp.where(first, flags, jnp.maximum(c0_s[...], flags))
        sums_s[...] = jnp.where(first, psum, sums_s[...] + psum)

        @pl.when(t == tiles - 1)
        def _aux():
            mean = sums_s[...] * (1.0 / ntok)
            aux_ref[...] = jnp.sum(mean * jnp.log(mean * ne + 1e-09),
                                   axis=-1, keepdims=True)

    @pl.when(p == 1)
    def _pass1():
        probs = probs_s[pl.ds(t * tile, tile), :]
        p0, e0, p1, e1 = _top2(probs, tile, ne)
        s = p0 + p1
        p0n = p0 / s
        p1n = p1 / s
        c0 = c0_s[...]
        slot1 = jnp.sum(jnp.where(eidx == e1, c0, 0.0), axis=-1,
                        keepdims=True).astype(jnp.int32)
        slotmat = jnp.where(eidx == e0, 0,
                            jnp.where(eidx == e1, slot1, -1))
        valmat = jnp.where(eidx == e0, p0n,
                           jnp.where(eidx == e1, p1n, 0.0))
        slot = t % 2

        def _ccopy(src, dst_ref, s, step):
            return pltpu.make_async_copy(
                src.at[s],
                dst_ref.at[pl.ds(step * tile, tile), :, pl.ds(0, _LEAD)],
                sem_c.at[s])

        @pl.when(t >= 2)
        def _reuse_wait():
            _ccopy(dbuf, disp_ref, slot, t - 2).wait()
            _ccopy(cbuf, comb_ref, slot, t - 2).wait()

        siota = jax.lax.broadcasted_iota(jnp.int32, (tile, ne, _LEAD), 2)
        hit = siota == slotmat[:, :, None]
        dbuf[slot] = hit.astype(jnp.float32)
        cbuf[slot] = jnp.where(hit, valmat[:, :, None], 0.0)
        _ccopy(dbuf, disp_ref, slot, t).start()
        _ccopy(cbuf, comb_ref, slot, t).start()
        probs_ref[...] = probs

        @pl.when(t == tiles - 1)
        def _drain():
            _ccopy(dbuf, disp_ref, 1 - slot, t - 1).wait()
            _ccopy(cbuf, comb_ref, 1 - slot, t - 1).wait()
            _ccopy(dbuf, disp_ref, slot, t).wait()
            _ccopy(cbuf, comb_ref, slot, t).wait()
            for qi in range(4):
                for _ in range(2 * tiles // 4):
                    _zcopy(disp_ref, 0, qi).wait()


def kernel(hidden_states, W1, b1, W2, b2):
    bb, ss, hh = hidden_states.shape
    ne = W2.shape[1]
    ntok = bb * ss
    cap = int(bb * ss * 1.5 * 2 / ne)
    x = hidden_states.reshape(ntok, hh)
    b1r = b1.reshape(1, hh)
    b2r = b2.reshape(1, ne)
    tile = 512
    tiles = ntok // tile

    body = functools.partial(_router_kernel, tile=tile, tiles=tiles,
                             ne=ne, cap=cap, ntok=ntok)

    disp, comb, probs, aux = pl.pallas_call(
        body,
        grid=(2, tiles),
        in_specs=[
            pl.BlockSpec((tile, hh), lambda p, t: (jnp.where(p == 0, t, 0), 0)),
            pl.BlockSpec((hh, hh), lambda p, t: (0, 0)),
            pl.BlockSpec((1, hh), lambda p, t: (0, 0)),
            pl.BlockSpec((hh, ne), lambda p, t: (0, 0)),
            pl.BlockSpec((1, ne), lambda p, t: (0, 0)),
        ],
        out_specs=[
            pl.BlockSpec(memory_space=pl.ANY),
            pl.BlockSpec(memory_space=pl.ANY),
            pl.BlockSpec((tile, ne), lambda p, t: (jnp.where(p == 1, t, 0), 0)),
            pl.BlockSpec((1, 1), lambda p, t: (0, 0)),
        ],
        out_shape=[
            jax.ShapeDtypeStruct((ntok, ne, cap), jnp.float32),
            jax.ShapeDtypeStruct((ntok, ne, cap), jnp.float32),
            jax.ShapeDtypeStruct((ntok, ne), jnp.float32),
            jax.ShapeDtypeStruct((1, 1), jnp.float32),
        ],
        scratch_shapes=[
            pltpu.VMEM((ntok, ne), jnp.float32),
            pltpu.VMEM((1, ne), jnp.float32),
            pltpu.VMEM((1, ne), jnp.float32),
            pltpu.VMEM((tile, ne, cap - _LEAD), jnp.float32),
            pltpu.VMEM((2, tile, ne, _LEAD), jnp.float32),
            pltpu.VMEM((2, tile, ne, _LEAD), jnp.float32),
            pltpu.SemaphoreType.DMA((4,)),
            pltpu.SemaphoreType.DMA((2,)),
        ],
        compiler_params=pltpu.CompilerParams(
            dimension_semantics=("arbitrary", "arbitrary")),
    )(x, W1, b1r, W2, b2r)

    return (disp.reshape(bb, ss, ne, cap),
            comb.reshape(bb, ss, ne, cap),
            probs.reshape(bb, ss, ne),
            aux.reshape(()))
